# baseline (speedup 1.0000x reference)
import jax
import jax.numpy as jnp
from jax import lax
from jax.experimental import pallas as pl
from jax.experimental.pallas import tpu as pltpu
from jax.experimental.pallas import tpu_sc as plsc

_B = 16 * 2048
_SPLITS = (8192, 16384, 8192)
_EMB = (128, 256, 128, 512)
_D = 1024
_NW = 32
_CH = (128, 64, 128, 32)


def _sc_gather_body(bpw, x0, x1, x2, x3, t0, t1, t2, t3,
                    e0, e1, e2, e3,
                    ix0, ix1, ix2, ix3,
                    a01, b01, a1, b1, a3, b3,
                    gsa, gsb, wsa, wsb):
    wid = lax.axis_index("s") * 2 + lax.axis_index("c")
    base = wid * bpw

    for xi, ixv in ((x0, ix0), (x1, ix1), (x2, ix2), (x3, ix3)):
        pltpu.sync_copy(xi.at[pl.ds(base, bpw)], ixv)

    def run_table(ixv, ti, ei, bufA, bufB, ch):
        npairs = bpw // ch // 2

        def gather(j, buf, sem):
            pltpu.async_copy(ti.at[ixv.at[pl.ds(j * ch, ch)]], buf, sem)

        def wait_gather(buf, sem):
            pltpu.make_async_copy(ti.at[ixv.at[pl.ds(0, ch)]], buf, sem).wait()

        def write(j, buf, sem):
            pltpu.async_copy(buf, ei.at[pl.ds(base + j * ch, ch)], sem)

        def wait_write(buf, sem):
            pltpu.make_async_copy(buf, ei.at[pl.ds(base, ch)], sem).wait()

        gather(0, bufA, gsa)
        gather(1, bufB, gsb)

        def body(k, carry):
            j = 2 * k
            wait_gather(bufA, gsa)
            write(j, bufA, wsa)
            wait_gather(bufB, gsb)
            write(j + 1, bufB, wsb)

            @pl.when(k < npairs - 1)
            def _():
                wait_write(bufA, wsa)
                gather(j + 2, bufA, gsa)
                wait_write(bufB, wsb)
                gather(j + 3, bufB, gsb)

            return carry

        lax.fori_loop(0, npairs, body, 0)
        wait_write(bufA, wsa)
        wait_write(bufB, wsb)

    run_table(ix0, t0, e0, a01, b01, _CH[0])
    run_table(ix2, t2, e2, a01, b01, _CH[2])
    run_table(ix1, t1, e1, a1, b1, _CH[1])
    run_table(ix3, t3, e3, a3, b3, _CH[3])


def _sc_gather(bc, x0, x1, x2, x3, t0, t1, t2, t3):
    import functools
    bpw = bc // _NW
    mesh = plsc.VectorSubcoreMesh(core_axis_name="c", subcore_axis_name="s")
    k = pl.kernel(
        functools.partial(_sc_gather_body, bpw),
        out_type=[jax.ShapeDtypeStruct((bc, e), jnp.float32) for e in _EMB],
        mesh=mesh,
        scratch_types=[
            pltpu.VMEM((bpw,), jnp.int32),
            pltpu.VMEM((bpw,), jnp.int32),
            pltpu.VMEM((bpw,), jnp.int32),
            pltpu.VMEM((bpw,), jnp.int32),
            pltpu.VMEM((_CH[0], _EMB[0]), jnp.float32),
            pltpu.VMEM((_CH[0], _EMB[0]), jnp.float32),
            pltpu.VMEM((_CH[1], _EMB[1]), jnp.float32),
            pltpu.VMEM((_CH[1], _EMB[1]), jnp.float32),
            pltpu.VMEM((_CH[3], _EMB[3]), jnp.float32),
            pltpu.VMEM((_CH[3], _EMB[3]), jnp.float32),
            pltpu.SemaphoreType.DMA,
            pltpu.SemaphoreType.DMA,
            pltpu.SemaphoreType.DMA,
            pltpu.SemaphoreType.DMA,
        ],
    )
    return k(x0, x1, x2, x3, t0, t1, t2, t3)


def _tc_matmul_chunk(offset, bc, e0, e1, e2, e3, W, b, prev):
    bm = 2048
    nb = bc // bm
    ob = offset // bm
    first = prev is None

    def body(*refs):
        e0r, e1r, e2r, e3r, w, bias = refs[:6]
        out = refs[-1]
        acc = jnp.dot(e0r[...], w[0:128, :], preferred_element_type=jnp.float32)
        acc = acc + jnp.dot(e1r[...], w[128:384, :],
                            preferred_element_type=jnp.float32)
        acc = acc + jnp.dot(e2r[...], w[384:512, :],
                            preferred_element_type=jnp.float32)
        acc = acc + jnp.dot(e3r[...], w[512:1024, :],
                            preferred_element_type=jnp.float32)
        out[...] = acc + bias[...]

    in_specs = [
        pl.BlockSpec((bm, _EMB[0]), lambda i: (i, 0)),
        pl.BlockSpec((bm, _EMB[1]), lambda i: (i, 0)),
        pl.BlockSpec((bm, _EMB[2]), lambda i: (i, 0)),
        pl.BlockSpec((bm, _EMB[3]), lambda i: (i, 0)),
        pl.BlockSpec((sum(_EMB), _D), lambda i: (0, 0)),
        pl.BlockSpec((1, _D), lambda i: (0, 0)),
    ]
    args = [e0, e1, e2, e3, W, b.reshape(1, _D)]
    aliases = {}
    if not first:
        in_specs.append(pl.BlockSpec(memory_space=pl.ANY))
        args.append(prev)
        aliases = {6: 0}
    return pl.pallas_call(
        body,
        grid=(nb,),
        in_specs=in_specs,
        out_specs=pl.BlockSpec((bm, _D), lambda i, ob=ob: (ob + i, 0)),
        out_shape=jax.ShapeDtypeStruct((_B, _D), jnp.float32),
        input_output_aliases=aliases,
        compiler_params=pltpu.CompilerParams(
            dimension_semantics=("arbitrary",),
        ),
    )(*args)


def kernel(x, t0, t1, t2, t3, W, b):
    lead = x.shape[:-1]
    xr = x.reshape(-1, 4).astype(jnp.int32)
    es, offs = [], []
    off = 0
    for bc in _SPLITS:
        xc = jax.lax.slice_in_dim(xr, off, off + bc, axis=0)
        es.append(_sc_gather(
            bc, xc[:, 0], xc[:, 1], xc[:, 2], xc[:, 3], t0, t1, t2, t3))
        offs.append(off)
        off += bc
    out = None
    for bc, off, e in zip(_SPLITS, offs, es):
        out = _tc_matmul_chunk(off, bc, *e, W, b, out)
    return out.reshape(*lead, _D)


# speedup vs baseline: 1.0287x; 1.0287x over previous
import functools

import jax
import jax.numpy as jnp
from jax import lax
from jax.experimental import pallas as pl
from jax.experimental.pallas import tpu as pltpu
from jax.experimental.pallas import tpu_sc as plsc

_B = 16 * 2048
_SPLITS = (16384, 16384)
_EMB = (128, 256, 128, 512)
_D = 1024
_NW = 32
_CH = (128, 64, 128, 32)


def _sc_gather_body(bpw, x0, x1, x2, x3, t0, t1, t2, t3,
                    e0, e1, e2, e3,
                    ix0, ix1, ix2, ix3,
                    a01, b01, a1, b1, a3, b3,
                    gsa, gsb, wsa, wsb):
    wid = lax.axis_index("s") * 2 + lax.axis_index("c")
    base = wid * bpw

    for xi, ixv in ((x0, ix0), (x1, ix1), (x2, ix2), (x3, ix3)):
        pltpu.sync_copy(xi.at[pl.ds(base, bpw)], ixv)

    def run_table(ixv, ti, ei, bufA, bufB, ch):
        npairs = bpw // ch // 2

        def gather(j, buf, sem):
            pltpu.async_copy(ti.at[ixv.at[pl.ds(j * ch, ch)]], buf, sem)

        def wait_gather(buf, sem):
            pltpu.make_async_copy(ti.at[ixv.at[pl.ds(0, ch)]], buf, sem).wait()

        def write(j, buf, sem):
            pltpu.async_copy(buf, ei.at[pl.ds(base + j * ch, ch)], sem)

        def wait_write(buf, sem):
            pltpu.make_async_copy(buf, ei.at[pl.ds(base, ch)], sem).wait()

        gather(0, bufA, gsa)
        gather(1, bufB, gsb)

        def body(k, carry):
            j = 2 * k
            wait_gather(bufA, gsa)
            write(j, bufA, wsa)
            wait_gather(bufB, gsb)
            write(j + 1, bufB, wsb)

            @pl.when(k < npairs - 1)
            def _():
                wait_write(bufA, wsa)
                gather(j + 2, bufA, gsa)
                wait_write(bufB, wsb)
                gather(j + 3, bufB, gsb)

            return carry

        lax.fori_loop(0, npairs, body, 0)
        wait_write(bufA, wsa)
        wait_write(bufB, wsb)

    run_table(ix0, t0, e0, a01, b01, _CH[0])
    run_table(ix2, t2, e2, a01, b01, _CH[2])
    run_table(ix1, t1, e1, a1, b1, _CH[1])
    run_table(ix3, t3, e3, a3, b3, _CH[3])


def _sc_gather(bc, x0, x1, x2, x3, t0, t1, t2, t3):
    bpw = bc // _NW
    mesh = plsc.VectorSubcoreMesh(core_axis_name="c", subcore_axis_name="s")
    k = pl.kernel(
        functools.partial(_sc_gather_body, bpw),
        out_type=[jax.ShapeDtypeStruct((bc, e), jnp.float32) for e in _EMB],
        mesh=mesh,
        scratch_types=[
            pltpu.VMEM((bpw,), jnp.int32),
            pltpu.VMEM((bpw,), jnp.int32),
            pltpu.VMEM((bpw,), jnp.int32),
            pltpu.VMEM((bpw,), jnp.int32),
            pltpu.VMEM((_CH[0], _EMB[0]), jnp.float32),
            pltpu.VMEM((_CH[0], _EMB[0]), jnp.float32),
            pltpu.VMEM((_CH[1], _EMB[1]), jnp.float32),
            pltpu.VMEM((_CH[1], _EMB[1]), jnp.float32),
            pltpu.VMEM((_CH[3], _EMB[3]), jnp.float32),
            pltpu.VMEM((_CH[3], _EMB[3]), jnp.float32),
            pltpu.SemaphoreType.DMA,
            pltpu.SemaphoreType.DMA,
            pltpu.SemaphoreType.DMA,
            pltpu.SemaphoreType.DMA,
        ],
    )
    return k(x0, x1, x2, x3, t0, t1, t2, t3)


def _tc_matmul_chunk(offset, bc, e0, e1, e2, e3, W, b, prev):
    bm = 2048
    nb = bc // bm
    ob = offset // bm
    first = prev is None

    def body(*refs):
        e0r, e1r, e2r, e3r, w, bias = refs[:6]
        out = refs[-1]
        acc = jnp.dot(e0r[...], w[0:128, :], preferred_element_type=jnp.float32)
        acc = acc + jnp.dot(e1r[...], w[128:384, :],
                            preferred_element_type=jnp.float32)
        acc = acc + jnp.dot(e2r[...], w[384:512, :],
                            preferred_element_type=jnp.float32)
        acc = acc + jnp.dot(e3r[...], w[512:1024, :],
                            preferred_element_type=jnp.float32)
        out[...] = acc + bias[...]

    in_specs = [
        pl.BlockSpec((bm, _EMB[0]), lambda i: (i, 0)),
        pl.BlockSpec((bm, _EMB[1]), lambda i: (i, 0)),
        pl.BlockSpec((bm, _EMB[2]), lambda i: (i, 0)),
        pl.BlockSpec((bm, _EMB[3]), lambda i: (i, 0)),
        pl.BlockSpec((sum(_EMB), _D), lambda i: (0, 0)),
        pl.BlockSpec((1, _D), lambda i: (0, 0)),
    ]
    args = [e0, e1, e2, e3, W, b.reshape(1, _D)]
    aliases = {}
    if not first:
        in_specs.append(pl.BlockSpec(memory_space=pl.ANY))
        args.append(prev)
        aliases = {6: 0}
    return pl.pallas_call(
        body,
        grid=(nb,),
        in_specs=in_specs,
        out_specs=pl.BlockSpec((bm, _D), lambda i, ob=ob: (ob + i, 0)),
        out_shape=jax.ShapeDtypeStruct((_B, _D), jnp.float32),
        input_output_aliases=aliases,
        compiler_params=pltpu.CompilerParams(
            dimension_semantics=("arbitrary",),
        ),
    )(*args)


def kernel(x, t0, t1, t2, t3, W, b):
    lead = x.shape[:-1]
    xr = x.reshape(-1, 4).astype(jnp.int32)
    es, offs = [], []
    off = 0
    for bc in _SPLITS:
        xc = jax.lax.slice_in_dim(xr, off, off + bc, axis=0)
        es.append(_sc_gather(
            bc, xc[:, 0], xc[:, 1], xc[:, 2], xc[:, 3], t0, t1, t2, t3))
        offs.append(off)
        off += bc
    out = None
    for bc, off, e in zip(_SPLITS, offs, es):
        out = _tc_matmul_chunk(off, bc, *e, W, b, out)
    return out.reshape(*lead, _D)


# speedup vs baseline: 1.0528x; 1.0235x over previous
import functools

import jax
import jax.numpy as jnp
from jax import lax
from jax.experimental import pallas as pl
from jax.experimental.pallas import tpu as pltpu
from jax.experimental.pallas import tpu_sc as plsc

_B = 16 * 2048
_SPLITS = (16384, 16384)
_EMB = (128, 256, 128, 512)
_D = 1024
_NW = 32
_CH = (128, 64, 128, 32)


def _sc_gather_body(bpw, x0, x1, x2, x3, t0, t1, t2, t3,
                    e0, e1, e2, e3,
                    ix0, ix1, ix2, ix3,
                    a01, b01, a1, b1, a3, b3,
                    gsa, gsb, ws01, ws1, ws3):
    wid = lax.axis_index("s") * 2 + lax.axis_index("c")
    base = wid * bpw

    for xi, ixv in ((x0, ix0), (x1, ix1), (x2, ix2), (x3, ix3)):
        pltpu.async_copy(xi.at[pl.ds(base, bpw)], ixv, gsa)
    for xi, ixv in ((x0, ix0), (x1, ix1), (x2, ix2), (x3, ix3)):
        pltpu.make_async_copy(xi.at[pl.ds(base, bpw)], ixv, gsa).wait()

    def run_table(ixv, ti, ei, bufA, bufB, ch, ws, drain):
        npairs = bpw // ch // 2
        wsa, wsb = ws

        def gather(j, buf, sem):
            pltpu.async_copy(ti.at[ixv.at[pl.ds(j * ch, ch)]], buf, sem)

        def wait_gather(buf, sem):
            pltpu.make_async_copy(ti.at[ixv.at[pl.ds(0, ch)]], buf, sem).wait()

        def write(j, buf, sem):
            pltpu.async_copy(buf, ei.at[pl.ds(base + j * ch, ch)], sem)

        def wait_write(buf, sem):
            pltpu.make_async_copy(buf, ei.at[pl.ds(base, ch)], sem).wait()

        if drain is not None:
            drain()
        gather(0, bufA, gsa)
        gather(1, bufB, gsb)

        def body(k, carry):
            j = 2 * k
            wait_gather(bufA, gsa)
            write(j, bufA, wsa)
            wait_gather(bufB, gsb)
            write(j + 1, bufB, wsb)

            @pl.when(k < npairs - 1)
            def _():
                wait_write(bufA, wsa)
                gather(j + 2, bufA, gsa)
                wait_write(bufB, wsb)
                gather(j + 3, bufB, gsb)

            return carry

        lax.fori_loop(0, npairs, body, 0)

        def drain_this():
            wait_write(bufA, wsa)
            wait_write(bufB, wsb)
        return drain_this

    d0 = run_table(ix0, t0, e0, a01, b01, _CH[0], ws01, None)
    d1 = run_table(ix1, t1, e1, a1, b1, _CH[1], ws1, None)
    d2 = run_table(ix2, t2, e2, a01, b01, _CH[2], ws01, d0)
    d3 = run_table(ix3, t3, e3, a3, b3, _CH[3], ws3, None)
    d1()
    d2()
    d3()


def _sc_gather(bc, x0, x1, x2, x3, t0, t1, t2, t3):
    bpw = bc // _NW
    mesh = plsc.VectorSubcoreMesh(core_axis_name="c", subcore_axis_name="s")
    k = pl.kernel(
        functools.partial(_sc_gather_body, bpw),
        out_type=[jax.ShapeDtypeStruct((bc, e), jnp.float32) for e in _EMB],
        mesh=mesh,
        scratch_types=[
            pltpu.VMEM((bpw,), jnp.int32),
            pltpu.VMEM((bpw,), jnp.int32),
            pltpu.VMEM((bpw,), jnp.int32),
            pltpu.VMEM((bpw,), jnp.int32),
            pltpu.VMEM((_CH[0], _EMB[0]), jnp.float32),
            pltpu.VMEM((_CH[0], _EMB[0]), jnp.float32),
            pltpu.VMEM((_CH[1], _EMB[1]), jnp.float32),
            pltpu.VMEM((_CH[1], _EMB[1]), jnp.float32),
            pltpu.VMEM((_CH[3], _EMB[3]), jnp.float32),
            pltpu.VMEM((_CH[3], _EMB[3]), jnp.float32),
            pltpu.SemaphoreType.DMA,
            pltpu.SemaphoreType.DMA,
            (pltpu.SemaphoreType.DMA, pltpu.SemaphoreType.DMA),
            (pltpu.SemaphoreType.DMA, pltpu.SemaphoreType.DMA),
            (pltpu.SemaphoreType.DMA, pltpu.SemaphoreType.DMA),
        ],
    )
    return k(x0, x1, x2, x3, t0, t1, t2, t3)


def _tc_matmul_chunk(offset, bc, e0, e1, e2, e3, W, b, prev):
    bm = 2048
    nb = bc // bm
    ob = offset // bm
    first = prev is None

    def body(*refs):
        e0r, e1r, e2r, e3r, w, bias = refs[:6]
        out = refs[-1]
        acc = jnp.dot(e0r[...], w[0:128, :], preferred_element_type=jnp.float32)
        acc = acc + jnp.dot(e1r[...], w[128:384, :],
                            preferred_element_type=jnp.float32)
        acc = acc + jnp.dot(e2r[...], w[384:512, :],
                            preferred_element_type=jnp.float32)
        acc = acc + jnp.dot(e3r[...], w[512:1024, :],
                            preferred_element_type=jnp.float32)
        out[...] = acc + bias[...]

    in_specs = [
        pl.BlockSpec((bm, _EMB[0]), lambda i: (i, 0)),
        pl.BlockSpec((bm, _EMB[1]), lambda i: (i, 0)),
        pl.BlockSpec((bm, _EMB[2]), lambda i: (i, 0)),
        pl.BlockSpec((bm, _EMB[3]), lambda i: (i, 0)),
        pl.BlockSpec((sum(_EMB), _D), lambda i: (0, 0)),
        pl.BlockSpec((1, _D), lambda i: (0, 0)),
    ]
    args = [e0, e1, e2, e3, W, b.reshape(1, _D)]
    aliases = {}
    if not first:
        in_specs.append(pl.BlockSpec(memory_space=pl.ANY))
        args.append(prev)
        aliases = {6: 0}
    return pl.pallas_call(
        body,
        grid=(nb,),
        in_specs=in_specs,
        out_specs=pl.BlockSpec((bm, _D), lambda i, ob=ob: (ob + i, 0)),
        out_shape=jax.ShapeDtypeStruct((_B, _D), jnp.float32),
        input_output_aliases=aliases,
        compiler_params=pltpu.CompilerParams(
            dimension_semantics=("arbitrary",),
        ),
    )(*args)


def kernel(x, t0, t1, t2, t3, W, b):
    lead = x.shape[:-1]
    xr = x.reshape(-1, 4).astype(jnp.int32)
    es, offs = [], []
    off = 0
    for bc in _SPLITS:
        xc = jax.lax.slice_in_dim(xr, off, off + bc, axis=0)
        es.append(_sc_gather(
            bc, xc[:, 0], xc[:, 1], xc[:, 2], xc[:, 3], t0, t1, t2, t3))
        offs.append(off)
        off += bc
    out = None
    for bc, off, e in zip(_SPLITS, offs, es):
        out = _tc_matmul_chunk(off, bc, *e, W, b, out)
    return out.reshape(*lead, _D)
